# baseline (speedup 1.0000x reference)
import functools

import jax
import jax.numpy as jnp
from jax import lax
from jax.experimental import pallas as pl
from jax.experimental.pallas import tpu as pltpu
from jax.experimental.pallas import tpu_sc as plsc

ALPHA = 0.1
CHUNK = 128
PAD_ROWS = 8


def _sc_segment_partials(inp, src3d, dst3d, N):
    D = inp.shape[1]
    NCH = src3d.shape[0]
    info = plsc.get_sparse_core_info()
    NC, NS = info.num_cores, info.num_subcores
    NW = NC * NS
    CPT = NCH // NW
    NG = CPT // 2
    NP = N + PAD_ROWS
    rpt = ((NP // NS) // 8) * 8
    extra = NP - NS * rpt
    n_full = rpt // CHUNK
    n_rem = rpt % CHUNK

    mesh = plsc.VectorSubcoreMesh(core_axis_name="c", subcore_axis_name="s")

    @functools.partial(
        pl.kernel,
        out_type=jax.ShapeDtypeStruct((NC, N, D), jnp.float32),
        mesh=mesh,
        scratch_types=[
            pltpu.VMEM_SHARED((NP, D), jnp.float32),
            pltpu.VMEM((CHUNK,), jnp.int32),
            pltpu.VMEM((CHUNK,), jnp.int32),
            pltpu.VMEM((CHUNK,), jnp.int32),
            pltpu.VMEM((CHUNK,), jnp.int32),
            pltpu.VMEM((CHUNK, D), jnp.float32),
            pltpu.VMEM((CHUNK, D), jnp.float32),
            pltpu.SemaphoreType.DMA,
            pltpu.SemaphoreType.DMA,
        ],
    )
    def k(inp_hbm, src_hbm, dst_hbm, out_hbm, acc, si0, si1, di0, di1,
          rows0, rows1, sem0, sem1):
        c = lax.axis_index("c")
        s = lax.axis_index("s")
        w = s * NC + c
        bufs = ((si0, di0, rows0, sem0), (si1, di1, rows1, sem1))

        zero = jnp.zeros((16,), jnp.float32)

        def zero_body(i, carry):
            for jcol in range(D // 16):
                rows0[i, pl.ds(jcol * 16, 16)] = zero
            return carry

        lax.fori_loop(0, CHUNK, zero_body, 0)
        r0 = s * rpt
        for j in range(n_full):
            pltpu.sync_copy(rows0, acc.at[pl.ds(r0 + j * CHUNK, CHUNK)])
        if n_rem:
            pltpu.sync_copy(
                rows0.at[pl.ds(0, n_rem)],
                acc.at[pl.ds(r0 + n_full * CHUNK, n_rem)],
            )

        @pl.when(s == NS - 1)
        def _():
            if extra:
                pltpu.sync_copy(
                    rows0.at[pl.ds(0, extra)],
                    acc.at[pl.ds(NS * rpt, extra)],
                )

        plsc.subcore_barrier()

        my_base = w * CPT

        def edge_body(j, carry):
            ch = my_base + j
            pltpu.sync_copy(src_hbm.at[ch, 0], si0)
            pltpu.sync_copy(dst_hbm.at[ch, 0], di0)
            pltpu.async_copy(inp_hbm.at[si0], rows0, sem0).wait()
            pltpu.sync_copy(rows0, acc.at[di0], add=True)
            return carry

        my_n = jnp.where(w < NW, CPT, 0)
        lax.fori_loop(0, my_n, edge_body, 0)
        plsc.subcore_barrier()

        def copy_out(row, nrows, buf):
            pltpu.sync_copy(acc.at[pl.ds(row, nrows)], buf.at[pl.ds(0, nrows)])
            pltpu.sync_copy(
                buf.at[pl.ds(0, nrows)], out_hbm.at[c, pl.ds(row, nrows)]
            )

        for j in range(n_full):
            copy_out(r0 + j * CHUNK, CHUNK, bufs[j % 2][2])
        if n_rem:
            copy_out(r0 + n_full * CHUNK, n_rem, rows0)

        n_tail = N - NS * rpt
        if n_tail > 0:

            @pl.when(s == NS - 1)
            def _():
                copy_out(NS * rpt, n_tail, rows1)

    return k(inp, src3d, dst3d)


def _blend(partials, input_emb):
    N, D = input_emb.shape
    BR = 1000
    grid = (N // BR,)

    def body(p0_ref, p1_ref, emb_ref, o_ref):
        o_ref[...] = (1.0 - ALPHA) * (p0_ref[...] + p1_ref[...]) + ALPHA * emb_ref[...]

    spec = pl.BlockSpec((BR, D), lambda i: (i, 0))
    return pl.pallas_call(
        body,
        grid=grid,
        in_specs=[spec, spec, spec],
        out_specs=spec,
        out_shape=jax.ShapeDtypeStruct((N, D), jnp.float32),
    )(partials[0], partials[1], input_emb)


def kernel(input, edge_index, input_emb, W):
    N = input.shape[0]
    E = edge_index.shape[1]
    NW = 32
    cpt = -(-E // (CHUNK * NW))
    cpt += cpt % 2
    e_pad = cpt * NW * CHUNK
    ei = edge_index.astype(jnp.int32)
    src = jnp.concatenate([ei[1], jnp.zeros((e_pad - E,), jnp.int32)])
    dst = jnp.concatenate([ei[0], jnp.full((e_pad - E,), N, jnp.int32)])
    src3d = src.reshape(-1, 1, CHUNK)
    dst3d = dst.reshape(-1, 1, CHUNK)
    partials = _sc_segment_partials(input, src3d, dst3d, N)
    return _blend(partials, input_emb)


# speedup vs baseline: 1.0030x; 1.0030x over previous
import functools

import jax
import jax.numpy as jnp
from jax import lax
from jax.experimental import pallas as pl
from jax.experimental.pallas import tpu as pltpu
from jax.experimental.pallas import tpu_sc as plsc

ALPHA = 0.1
CHUNK = 128
PAD_ROWS = 128


def _sc_segment_partials(inp, src3d, dst3d, N):
    D = inp.shape[1]
    NCH = src3d.shape[0]
    info = plsc.get_sparse_core_info()
    NC, NS = info.num_cores, info.num_subcores
    NW = NC * NS
    CPT = NCH // NW
    NG = CPT // 2
    NP = N + PAD_ROWS
    rpt = ((N // NS) // 8) * 8
    extra = NP - NS * rpt
    n_full = rpt // CHUNK
    n_rem = rpt % CHUNK

    mesh = plsc.VectorSubcoreMesh(core_axis_name="c", subcore_axis_name="s")

    @functools.partial(
        pl.kernel,
        out_type=jax.ShapeDtypeStruct((NC, N, D), jnp.float32),
        mesh=mesh,
        scratch_types=[
            pltpu.VMEM_SHARED((NP, D), jnp.float32),
            pltpu.VMEM((CHUNK,), jnp.int32),
            pltpu.VMEM((CHUNK,), jnp.int32),
            pltpu.VMEM((CHUNK,), jnp.int32),
            pltpu.VMEM((CHUNK,), jnp.int32),
            pltpu.VMEM((CHUNK, D), jnp.float32),
            pltpu.VMEM((CHUNK, D), jnp.float32),
            pltpu.SemaphoreType.DMA,
            pltpu.SemaphoreType.DMA,
        ],
    )
    def k(inp_hbm, src_hbm, dst_hbm, out_hbm, acc, si0, si1, di0, di1,
          rows0, rows1, sem0, sem1):
        c = lax.axis_index("c")
        s = lax.axis_index("s")
        w = s * NC + c
        bufs = ((si0, di0, rows0, sem0), (si1, di1, rows1, sem1))

        zero = jnp.zeros((16,), jnp.float32)

        def zero_body(i, carry):
            for jcol in range(D // 16):
                rows0[i, pl.ds(jcol * 16, 16)] = zero
            return carry

        lax.fori_loop(0, CHUNK, zero_body, 0)
        r0 = s * rpt
        for j in range(n_full):
            pltpu.sync_copy(rows0, acc.at[pl.ds(r0 + j * CHUNK, CHUNK)])
        if n_rem:
            pltpu.sync_copy(
                rows0.at[pl.ds(0, n_rem)],
                acc.at[pl.ds(r0 + n_full * CHUNK, n_rem)],
            )

        @pl.when(s == NS - 1)
        def _():
            done = 0
            while done < extra:
                step = min(CHUNK, extra - done)
                pltpu.sync_copy(
                    rows0.at[pl.ds(0, step)],
                    acc.at[pl.ds(NS * rpt + done, step)],
                )
                done += step

        plsc.subcore_barrier()

        my_base = w * CPT

        def edge_body(j, carry):
            ch = my_base + j
            pltpu.sync_copy(src_hbm.at[ch, 0], si0)
            pltpu.sync_copy(dst_hbm.at[ch, 0], di0)
            pltpu.async_copy(inp_hbm.at[si0], rows0, sem0).wait()
            pltpu.sync_copy(rows0, acc.at[di0], add=True)
            return carry

        my_n = jnp.where(w < NW, CPT, 0)
        lax.fori_loop(0, my_n, edge_body, 0)
        plsc.subcore_barrier()

        def copy_out(row, nrows, buf):
            pltpu.sync_copy(acc.at[pl.ds(row, nrows)], buf.at[pl.ds(0, nrows)])
            pltpu.sync_copy(
                buf.at[pl.ds(0, nrows)], out_hbm.at[c, pl.ds(row, nrows)]
            )

        for j in range(n_full):
            copy_out(r0 + j * CHUNK, CHUNK, bufs[j % 2][2])
        if n_rem:
            copy_out(r0 + n_full * CHUNK, n_rem, rows0)

        n_tail = N - NS * rpt
        if n_tail > 0:

            @pl.when(s == NS - 1)
            def _():
                copy_out(NS * rpt, n_tail, rows1)

    return k(inp, src3d, dst3d)


def _blend(partials, input_emb):
    N, D = input_emb.shape
    BR = 1000
    grid = (N // BR,)

    def body(p0_ref, p1_ref, emb_ref, o_ref):
        o_ref[...] = (1.0 - ALPHA) * (p0_ref[...] + p1_ref[...]) + ALPHA * emb_ref[...]

    spec = pl.BlockSpec((BR, D), lambda i: (i, 0))
    return pl.pallas_call(
        body,
        grid=grid,
        in_specs=[spec, spec, spec],
        out_specs=spec,
        out_shape=jax.ShapeDtypeStruct((N, D), jnp.float32),
    )(partials[0], partials[1], input_emb)


def kernel(input, edge_index, input_emb, W):
    N = input.shape[0]
    E = edge_index.shape[1]
    NW = 32
    cpt = -(-E // (CHUNK * NW))
    cpt += cpt % 2
    e_pad = cpt * NW * CHUNK
    ei = edge_index.astype(jnp.int32)
    n_pad = e_pad - E
    src = jnp.concatenate([ei[1], jnp.zeros((n_pad,), jnp.int32)])
    dst = jnp.concatenate(
        [ei[0], N + (jnp.arange(n_pad, dtype=jnp.int32) % PAD_ROWS)]
    )
    src3d = src.reshape(-1, 1, CHUNK)
    dst3d = dst.reshape(-1, 1, CHUNK)
    partials = _sc_segment_partials(input, src3d, dst3d, N)
    return _blend(partials, input_emb)


# speedup vs baseline: 2.4095x; 2.4024x over previous
import functools

import jax
import jax.numpy as jnp
from jax import lax
from jax.experimental import pallas as pl
from jax.experimental.pallas import tpu as pltpu
from jax.experimental.pallas import tpu_sc as plsc

ALPHA = 0.1
CHUNK = 128
PAD_ROWS = 128


def _sc_segment_partials(inp, src3d, dst3d, N):
    D = inp.shape[1]
    NCH = src3d.shape[0]
    info = plsc.get_sparse_core_info()
    NC, NS = info.num_cores, info.num_subcores
    NW = NC * NS
    CPT = NCH // NW
    NG = CPT // 2
    NP = N + PAD_ROWS
    rpt = ((N // NS) // 8) * 8
    extra = NP - NS * rpt
    n_full = rpt // CHUNK
    n_rem = rpt % CHUNK

    mesh = plsc.VectorSubcoreMesh(core_axis_name="c", subcore_axis_name="s")

    @functools.partial(
        pl.kernel,
        out_type=jax.ShapeDtypeStruct((NC, N, D), jnp.float32),
        mesh=mesh,
        scratch_types=[
            pltpu.VMEM_SHARED((NP, D), jnp.float32),
            pltpu.VMEM((CHUNK,), jnp.int32),
            pltpu.VMEM((CHUNK,), jnp.int32),
            pltpu.VMEM((CHUNK,), jnp.int32),
            pltpu.VMEM((CHUNK,), jnp.int32),
            pltpu.VMEM((CHUNK, D), jnp.float32),
            pltpu.VMEM((CHUNK, D), jnp.float32),
            pltpu.SemaphoreType.DMA,
            pltpu.SemaphoreType.DMA,
        ],
    )
    def k(inp_hbm, src_hbm, dst_hbm, out_hbm, acc, si0, si1, di0, di1,
          rows0, rows1, sem0, sem1):
        c = lax.axis_index("c")
        s = lax.axis_index("s")
        w = s * NC + c
        bufs = ((si0, di0, rows0, sem0), (si1, di1, rows1, sem1))

        zero = jnp.zeros((16,), jnp.float32)

        def zero_body(i, carry):
            for jcol in range(D // 16):
                rows0[i, pl.ds(jcol * 16, 16)] = zero
            return carry

        lax.fori_loop(0, CHUNK, zero_body, 0)
        r0 = s * rpt
        for j in range(n_full):
            pltpu.sync_copy(rows0, acc.at[pl.ds(r0 + j * CHUNK, CHUNK)])
        if n_rem:
            pltpu.sync_copy(
                rows0.at[pl.ds(0, n_rem)],
                acc.at[pl.ds(r0 + n_full * CHUNK, n_rem)],
            )

        @pl.when(s == NS - 1)
        def _():
            done = 0
            while done < extra:
                step = min(CHUNK, extra - done)
                pltpu.sync_copy(
                    rows0.at[pl.ds(0, step)],
                    acc.at[pl.ds(NS * rpt + done, step)],
                )
                done += step

        plsc.subcore_barrier()

        my_base = w * CPT

        def edge_body(j, carry):
            ch = my_base + j
            pltpu.sync_copy(src_hbm.at[ch, 0], si0)
            pltpu.sync_copy(dst_hbm.at[ch, 0], di0)
            pltpu.async_copy(inp_hbm.at[si0], rows0, sem0).wait()
            pltpu.sync_copy(rows0, acc.at[di0], add=True)
            return carry

        my_n = jnp.where(w < NW, CPT, 0)
        lax.fori_loop(0, my_n, edge_body, 0)
        plsc.subcore_barrier()

        def copy_out(row, nrows, buf):
            pltpu.sync_copy(acc.at[pl.ds(row, nrows)], buf.at[pl.ds(0, nrows)])
            pltpu.sync_copy(
                buf.at[pl.ds(0, nrows)], out_hbm.at[c, pl.ds(row, nrows)]
            )

        for j in range(n_full):
            copy_out(r0 + j * CHUNK, CHUNK, bufs[j % 2][2])
        if n_rem:
            copy_out(r0 + n_full * CHUNK, n_rem, rows0)

        n_tail = N - NS * rpt
        if n_tail > 0:

            @pl.when(s == NS - 1)
            def _():
                copy_out(NS * rpt, n_tail, rows1)

    return k(inp, src3d, dst3d)


def _blend(partials, input_emb):
    N, D = input_emb.shape
    BR = 1000
    grid = (N // BR,)

    def body(p0_ref, p1_ref, emb_ref, o_ref):
        o_ref[...] = (1.0 - ALPHA) * (p0_ref[...] + p1_ref[...]) + ALPHA * emb_ref[...]

    spec = pl.BlockSpec((BR, D), lambda i: (i, 0))
    return pl.pallas_call(
        body,
        grid=grid,
        in_specs=[spec, spec, spec],
        out_specs=spec,
        out_shape=jax.ShapeDtypeStruct((N, D), jnp.float32),
    )(partials[0], partials[1], input_emb)


def kernel(input, edge_index, input_emb, W):
    N = input.shape[0]
    E = edge_index.shape[1]
    NW = 32
    cpt = -(-E // (CHUNK * NW))
    cpt += cpt % 2
    e_pad = cpt * NW * CHUNK
    ei = edge_index.astype(jnp.int32)
    n_pad = e_pad - E
    pad_iota = jnp.arange(n_pad, dtype=jnp.int32)
    src = jnp.concatenate([ei[1], pad_iota % N])
    dst = jnp.concatenate([ei[0], N + (pad_iota % PAD_ROWS)])
    src3d = src.reshape(-1, 1, CHUNK)
    dst3d = dst.reshape(-1, 1, CHUNK)
    partials = _sc_segment_partials(input, src3d, dst3d, N)
    return _blend(partials, input_emb)


# speedup vs baseline: 3.6180x; 1.5016x over previous
import functools

import jax
import jax.numpy as jnp
from jax import lax
from jax.experimental import pallas as pl
from jax.experimental.pallas import tpu as pltpu
from jax.experimental.pallas import tpu_sc as plsc

ALPHA = 0.1
CHUNK = 128
PAD_ROWS = 128


def _sc_segment_partials(inp, src3d, dst3d, N):
    D = inp.shape[1]
    NCH = src3d.shape[0]
    info = plsc.get_sparse_core_info()
    NC, NS = info.num_cores, info.num_subcores
    NW = NC * NS
    CPT = NCH // NW
    NG = CPT // 2
    NP = N + PAD_ROWS
    rpt = ((N // NS) // 8) * 8
    extra = NP - NS * rpt
    n_full = rpt // CHUNK
    n_rem = rpt % CHUNK

    mesh = plsc.VectorSubcoreMesh(core_axis_name="c", subcore_axis_name="s")

    @functools.partial(
        pl.kernel,
        out_type=jax.ShapeDtypeStruct((NC, N, D), jnp.float32),
        mesh=mesh,
        scratch_types=[
            pltpu.VMEM_SHARED((NP, D), jnp.float32),
            pltpu.VMEM((CHUNK,), jnp.int32),
            pltpu.VMEM((CHUNK,), jnp.int32),
            pltpu.VMEM((CHUNK,), jnp.int32),
            pltpu.VMEM((CHUNK,), jnp.int32),
            pltpu.VMEM((CHUNK, D), jnp.float32),
            pltpu.VMEM((CHUNK, D), jnp.float32),
            pltpu.SemaphoreType.DMA,
            pltpu.SemaphoreType.DMA,
        ],
    )
    def k(inp_hbm, src_hbm, dst_hbm, out_hbm, acc, si0, si1, di0, di1,
          rows0, rows1, sem0, sem1):
        c = lax.axis_index("c")
        s = lax.axis_index("s")
        w = s * NC + c
        bufs = ((si0, di0, rows0, sem0), (si1, di1, rows1, sem1))

        zero = jnp.zeros((16,), jnp.float32)

        def zero_body(i, carry):
            for jcol in range(D // 16):
                rows0[i, pl.ds(jcol * 16, 16)] = zero
            return carry

        lax.fori_loop(0, CHUNK, zero_body, 0)
        r0 = s * rpt
        for j in range(n_full):
            pltpu.sync_copy(rows0, acc.at[pl.ds(r0 + j * CHUNK, CHUNK)])
        if n_rem:
            pltpu.sync_copy(
                rows0.at[pl.ds(0, n_rem)],
                acc.at[pl.ds(r0 + n_full * CHUNK, n_rem)],
            )

        @pl.when(s == NS - 1)
        def _():
            done = 0
            while done < extra:
                step = min(CHUNK, extra - done)
                pltpu.sync_copy(
                    rows0.at[pl.ds(0, step)],
                    acc.at[pl.ds(NS * rpt + done, step)],
                )
                done += step

        plsc.subcore_barrier()

        my_base = w * CPT
        pltpu.sync_copy(src_hbm.at[my_base, 0], si0)
        pltpu.sync_copy(dst_hbm.at[my_base, 0], di0)
        pltpu.async_copy(inp_hbm.at[si0], rows0, sem0)

        def edge_body(g, carry):
            j0 = my_base + 2 * g
            for b in range(2):
                si_b, di_b, rows_b, sem_b = bufs[b]
                si_n, di_n, rows_n, sem_n = bufs[1 - b]

                def start_next():
                    pltpu.sync_copy(src_hbm.at[j0 + b + 1, 0], si_n)
                    pltpu.sync_copy(dst_hbm.at[j0 + b + 1, 0], di_n)
                    pltpu.async_copy(inp_hbm.at[si_n], rows_n, sem_n)

                if b == 0:
                    start_next()
                else:
                    pl.when(g < NG - 1)(start_next)
                pltpu.make_async_copy(inp_hbm.at[si_b], rows_b, sem_b).wait()
                pltpu.sync_copy(rows_b, acc.at[di_b], add=True)
            return carry

        lax.fori_loop(0, NG, edge_body, 0)
        plsc.subcore_barrier()

        def copy_out(row, nrows, buf):
            pltpu.sync_copy(acc.at[pl.ds(row, nrows)], buf.at[pl.ds(0, nrows)])
            pltpu.sync_copy(
                buf.at[pl.ds(0, nrows)], out_hbm.at[c, pl.ds(row, nrows)]
            )

        for j in range(n_full):
            copy_out(r0 + j * CHUNK, CHUNK, bufs[j % 2][2])
        if n_rem:
            copy_out(r0 + n_full * CHUNK, n_rem, rows0)

        n_tail = N - NS * rpt
        if n_tail > 0:

            @pl.when(s == NS - 1)
            def _():
                copy_out(NS * rpt, n_tail, rows1)

    return k(inp, src3d, dst3d)


def _blend(partials, input_emb):
    N, D = input_emb.shape
    BR = 1000
    grid = (N // BR,)

    def body(p0_ref, p1_ref, emb_ref, o_ref):
        o_ref[...] = (1.0 - ALPHA) * (p0_ref[...] + p1_ref[...]) + ALPHA * emb_ref[...]

    spec = pl.BlockSpec((BR, D), lambda i: (i, 0))
    return pl.pallas_call(
        body,
        grid=grid,
        in_specs=[spec, spec, spec],
        out_specs=spec,
        out_shape=jax.ShapeDtypeStruct((N, D), jnp.float32),
    )(partials[0], partials[1], input_emb)


def kernel(input, edge_index, input_emb, W):
    N = input.shape[0]
    E = edge_index.shape[1]
    NW = 32
    cpt = -(-E // (CHUNK * NW))
    cpt += cpt % 2
    e_pad = cpt * NW * CHUNK
    ei = edge_index.astype(jnp.int32)
    n_pad = e_pad - E
    pad_iota = jnp.arange(n_pad, dtype=jnp.int32)
    src = jnp.concatenate([ei[1], pad_iota % N])
    dst = jnp.concatenate([ei[0], N + (pad_iota % PAD_ROWS)])
    src3d = src.reshape(-1, 1, CHUNK)
    dst3d = dst.reshape(-1, 1, CHUNK)
    partials = _sc_segment_partials(input, src3d, dst3d, N)
    return _blend(partials, input_emb)


# speedup vs baseline: 4.6716x; 1.2912x over previous
import functools

import jax
import jax.numpy as jnp
from jax import lax
from jax.experimental import pallas as pl
from jax.experimental.pallas import tpu as pltpu
from jax.experimental.pallas import tpu_sc as plsc

ALPHA = 0.1
CHUNK = 128
PAD_ROWS = 128


def _sc_segment_partials(inp, cidx3d, N):
    D = inp.shape[1]
    info = plsc.get_sparse_core_info()
    NC, NS = info.num_cores, info.num_subcores
    NW = NC * NS
    CPT = cidx3d.shape[1]
    NG = CPT // 2
    NP = N + PAD_ROWS
    rpt = ((N // NS) // 8) * 8
    extra = NP - NS * rpt
    n_full = rpt // CHUNK
    n_rem = rpt % CHUNK

    mesh = plsc.VectorSubcoreMesh(core_axis_name="c", subcore_axis_name="s")

    @functools.partial(
        pl.kernel,
        out_type=jax.ShapeDtypeStruct((NC, N, D), jnp.float32),
        mesh=mesh,
        scratch_types=[
            pltpu.VMEM_SHARED((NP, D), jnp.float32),
            pltpu.VMEM((CPT, CHUNK), jnp.int32),
            pltpu.VMEM((CHUNK,), jnp.int32),
            pltpu.VMEM((CHUNK,), jnp.int32),
            pltpu.VMEM((CHUNK,), jnp.int32),
            pltpu.VMEM((CHUNK,), jnp.int32),
            pltpu.VMEM((CHUNK, D), jnp.float32),
            pltpu.VMEM((CHUNK, D), jnp.float32),
            pltpu.SemaphoreType.DMA,
            pltpu.SemaphoreType.DMA,
        ],
    )
    def k(inp_hbm, cidx_hbm, out_hbm, acc, cidx,
          si0, si1, di0, di1, rows0, rows1, sem0, sem1):
        c = lax.axis_index("c")
        s = lax.axis_index("s")
        w = s * NC + c
        bufs = ((si0, di0, rows0, sem0), (si1, di1, rows1, sem1))

        zero = jnp.zeros((16,), jnp.float32)

        def zero_body(i, carry):
            for jcol in range(D // 16):
                rows0[i, pl.ds(jcol * 16, 16)] = zero
            return carry

        lax.fori_loop(0, CHUNK, zero_body, 0)
        r0 = s * rpt
        for j in range(n_full):
            pltpu.sync_copy(rows0, acc.at[pl.ds(r0 + j * CHUNK, CHUNK)])
        if n_rem:
            pltpu.sync_copy(
                rows0.at[pl.ds(0, n_rem)],
                acc.at[pl.ds(r0 + n_full * CHUNK, n_rem)],
            )

        @pl.when(s == NS - 1)
        def _():
            done = 0
            while done < extra:
                step = min(CHUNK, extra - done)
                pltpu.sync_copy(
                    rows0.at[pl.ds(0, step)],
                    acc.at[pl.ds(NS * rpt + done, step)],
                )
                done += step

        plsc.subcore_barrier()

        pltpu.sync_copy(cidx_hbm.at[w], cidx)

        def load_idx(j, si_b, di_b):
            for kk in range(CHUNK // 16):
                sl = pl.ds(kk * 16, 16)
                v = cidx[j, sl]
                si_b[sl] = v & 0xFFFF
                di_b[sl] = lax.shift_right_logical(v, 16)

        load_idx(0, si0, di0)
        pltpu.async_copy(inp_hbm.at[si0], rows0, sem0)

        def edge_body(g, carry):
            j0 = 2 * g
            for b in range(2):
                j = j0 + b
                si_b, di_b, rows_b, sem_b = bufs[b]
                si_n, di_n, rows_n, sem_n = bufs[1 - b]

                def start_next():
                    load_idx(j + 1, si_n, di_n)
                    pltpu.async_copy(inp_hbm.at[si_n], rows_n, sem_n)

                if b == 0:
                    start_next()
                else:
                    pl.when(g < NG - 1)(start_next)
                pltpu.make_async_copy(inp_hbm.at[si_b], rows_b, sem_b).wait()
                pltpu.sync_copy(rows_b, acc.at[di_b], add=True)
            return carry

        lax.fori_loop(0, NG, edge_body, 0)
        plsc.subcore_barrier()

        def copy_out(row, nrows, buf):
            pltpu.sync_copy(acc.at[pl.ds(row, nrows)], buf.at[pl.ds(0, nrows)])
            pltpu.sync_copy(
                buf.at[pl.ds(0, nrows)], out_hbm.at[c, pl.ds(row, nrows)]
            )

        for j in range(n_full):
            copy_out(r0 + j * CHUNK, CHUNK, bufs[j % 2][2])
        if n_rem:
            copy_out(r0 + n_full * CHUNK, n_rem, rows0)

        n_tail = N - NS * rpt
        if n_tail > 0:

            @pl.when(s == NS - 1)
            def _():
                copy_out(NS * rpt, n_tail, rows1)

    return k(inp, cidx3d)


def _blend(partials, input_emb):
    N, D = input_emb.shape
    BR = 1000
    grid = (N // BR,)

    def body(p0_ref, p1_ref, emb_ref, o_ref):
        o_ref[...] = (1.0 - ALPHA) * (p0_ref[...] + p1_ref[...]) + ALPHA * emb_ref[...]

    spec = pl.BlockSpec((BR, D), lambda i: (i, 0))
    return pl.pallas_call(
        body,
        grid=grid,
        in_specs=[spec, spec, spec],
        out_specs=spec,
        out_shape=jax.ShapeDtypeStruct((N, D), jnp.float32),
    )(partials[0], partials[1], input_emb)


def kernel(input, edge_index, input_emb, W):
    N = input.shape[0]
    E = edge_index.shape[1]
    NW = 32
    cpt = -(-E // (CHUNK * NW))
    cpt += cpt % 2
    e_pad = cpt * NW * CHUNK
    ei = edge_index.astype(jnp.int32)
    n_pad = e_pad - E
    pad_iota = jnp.arange(n_pad, dtype=jnp.int32)
    src = jnp.concatenate([ei[1], pad_iota % N])
    dst = jnp.concatenate([ei[0], N + (pad_iota % PAD_ROWS)])
    cidx3d = ((dst << 16) | src).reshape(NW, -1, CHUNK)
    partials = _sc_segment_partials(input, cidx3d, N)
    return _blend(partials, input_emb)


# speedup vs baseline: 4.7729x; 1.0217x over previous
import functools

import jax
import jax.numpy as jnp
from jax import lax
from jax.experimental import pallas as pl
from jax.experimental.pallas import tpu as pltpu
from jax.experimental.pallas import tpu_sc as plsc

ALPHA = 0.1
CHUNK = 128
PAD_ROWS = 128


def _sc_segment_partials(inp, cidx3d, N):
    D = inp.shape[1]
    info = plsc.get_sparse_core_info()
    NC, NS = info.num_cores, info.num_subcores
    NW = NC * NS
    CPT = cidx3d.shape[1]
    NG = CPT // 2
    NP = N + PAD_ROWS
    rpt = ((N // NS) // 8) * 8
    extra = NP - NS * rpt
    n_full = rpt // CHUNK
    n_rem = rpt % CHUNK

    mesh = plsc.VectorSubcoreMesh(core_axis_name="c", subcore_axis_name="s")

    @functools.partial(
        pl.kernel,
        out_type=jax.ShapeDtypeStruct((NC, N, D), jnp.float32),
        mesh=mesh,
        scratch_types=[
            pltpu.VMEM_SHARED((NP, D), jnp.float32),
            pltpu.VMEM((CPT, CHUNK), jnp.int32),
            pltpu.VMEM((CHUNK,), jnp.int32),
            pltpu.VMEM((CHUNK,), jnp.int32),
            pltpu.VMEM((CHUNK,), jnp.int32),
            pltpu.VMEM((CHUNK,), jnp.int32),
            pltpu.VMEM((CHUNK, D), jnp.float32),
            pltpu.VMEM((CHUNK, D), jnp.float32),
            pltpu.SemaphoreType.DMA,
            pltpu.SemaphoreType.DMA,
        ],
    )
    def k(inp_hbm, cidx_hbm, out_hbm, acc, cidx,
          si0, si1, di0, di1, rows0, rows1, sem0, sem1):
        c = lax.axis_index("c")
        s = lax.axis_index("s")
        w = s * NC + c
        bufs = ((si0, di0, rows0, sem0), (si1, di1, rows1, sem1))

        cidx_load = pltpu.async_copy(cidx_hbm.at[w], cidx, sem1)
        zero = jnp.zeros((16,), jnp.float32)

        def zero_body(i, carry):
            for jcol in range(D // 16):
                rows0[i, pl.ds(jcol * 16, 16)] = zero
            return carry

        lax.fori_loop(0, CHUNK, zero_body, 0)
        r0 = s * rpt
        zcs = []
        for j in range(n_full):
            zcs.append(
                pltpu.async_copy(rows0, acc.at[pl.ds(r0 + j * CHUNK, CHUNK)], sem0)
            )
        if n_rem:
            zcs.append(
                pltpu.async_copy(
                    rows0.at[pl.ds(0, n_rem)],
                    acc.at[pl.ds(r0 + n_full * CHUNK, n_rem)],
                    sem0,
                )
            )

        @pl.when(s == NS - 1)
        def _():
            done = 0
            while done < extra:
                step = min(CHUNK, extra - done)
                pltpu.sync_copy(
                    rows0.at[pl.ds(0, step)],
                    acc.at[pl.ds(NS * rpt + done, step)],
                )
                done += step

        for d in zcs:
            d.wait()
        cidx_load.wait()
        plsc.subcore_barrier()


        def load_idx(j, si_b, di_b):
            for kk in range(CHUNK // 16):
                sl = pl.ds(kk * 16, 16)
                v = cidx[j, sl]
                si_b[sl] = v & 0xFFFF
                di_b[sl] = lax.shift_right_logical(v, 16)

        load_idx(0, si0, di0)
        pltpu.async_copy(inp_hbm.at[si0], rows0, sem0)

        def edge_body(g, carry):
            j0 = 2 * g
            for b in range(2):
                j = j0 + b
                si_b, di_b, rows_b, sem_b = bufs[b]
                si_n, di_n, rows_n, sem_n = bufs[1 - b]

                def start_next():
                    load_idx(j + 1, si_n, di_n)
                    pltpu.async_copy(inp_hbm.at[si_n], rows_n, sem_n)

                if b == 0:
                    start_next()
                else:
                    pl.when(g < NG - 1)(start_next)
                pltpu.make_async_copy(inp_hbm.at[si_b], rows_b, sem_b).wait()
                pltpu.sync_copy(rows_b, acc.at[di_b], add=True)
            return carry

        lax.fori_loop(0, NG, edge_body, 0)
        plsc.subcore_barrier()

        wb = [(r0 + j * CHUNK, CHUNK) for j in range(n_full)]
        if n_rem:
            wb.append((r0 + n_full * CHUNK, n_rem))
        rbufs = (rows0, rows1)
        wsems = (sem0, sem1)
        prev = [None, None]
        for kk, (row, nrows) in enumerate(wb):
            b = kk % 2
            if prev[b] is not None:
                prow, pn = prev[b]
                pltpu.make_async_copy(
                    rbufs[b].at[pl.ds(0, pn)],
                    out_hbm.at[c, pl.ds(prow, pn)],
                    wsems[b],
                ).wait()
            pltpu.sync_copy(acc.at[pl.ds(row, nrows)], rbufs[b].at[pl.ds(0, nrows)])
            pltpu.async_copy(
                rbufs[b].at[pl.ds(0, nrows)],
                out_hbm.at[c, pl.ds(row, nrows)],
                wsems[b],
            )
            prev[b] = (row, nrows)
        for b in (0, 1):
            if prev[b] is not None:
                prow, pn = prev[b]
                pltpu.make_async_copy(
                    rbufs[b].at[pl.ds(0, pn)],
                    out_hbm.at[c, pl.ds(prow, pn)],
                    wsems[b],
                ).wait()

        n_tail = N - NS * rpt
        if n_tail > 0:

            @pl.when(s == NS - 1)
            def _():
                pltpu.sync_copy(
                    acc.at[pl.ds(NS * rpt, n_tail)], rows0.at[pl.ds(0, n_tail)]
                )
                pltpu.sync_copy(
                    rows0.at[pl.ds(0, n_tail)],
                    out_hbm.at[c, pl.ds(NS * rpt, n_tail)],
                )

    return k(inp, cidx3d)


def _blend(partials, input_emb):
    N, D = input_emb.shape
    BR = 1000
    grid = (N // BR,)

    def body(p0_ref, p1_ref, emb_ref, o_ref):
        o_ref[...] = (1.0 - ALPHA) * (p0_ref[...] + p1_ref[...]) + ALPHA * emb_ref[...]

    spec = pl.BlockSpec((BR, D), lambda i: (i, 0))
    return pl.pallas_call(
        body,
        grid=grid,
        in_specs=[spec, spec, spec],
        out_specs=spec,
        out_shape=jax.ShapeDtypeStruct((N, D), jnp.float32),
    )(partials[0], partials[1], input_emb)


def kernel(input, edge_index, input_emb, W):
    N = input.shape[0]
    E = edge_index.shape[1]
    NW = 32
    cpt = -(-E // (CHUNK * NW))
    cpt += cpt % 2
    e_pad = cpt * NW * CHUNK
    ei = edge_index.astype(jnp.int32)
    n_pad = e_pad - E
    pad_iota = jnp.arange(n_pad, dtype=jnp.int32)
    src = jnp.concatenate([ei[1], pad_iota % N])
    dst = jnp.concatenate([ei[0], N + (pad_iota % PAD_ROWS)])
    cidx3d = ((dst << 16) | src).reshape(NW, -1, CHUNK)
    partials = _sc_segment_partials(input, cidx3d, N)
    return _blend(partials, input_emb)
